# baseline (speedup 1.0000x reference)
import functools

import jax
import jax.numpy as jnp
from jax import lax
from jax.experimental import pallas as pl
from jax.experimental.pallas import tpu as pltpu
from jax.experimental.pallas import tpu_sc as plsc

N = 10000
K = 20
BIG = 1e30
CW = 256
NPAD = 10240


def _bdot(a, b):
    return jax.lax.dot_general(
        a.astype(jnp.bfloat16), b.astype(jnp.bfloat16),
        (((1,), (0,)), ((), ())), preferred_element_type=jnp.float32)


def _bn_apply_kernel(x_ref, m_ref, den_ref, g_ref, b_ref, y_ref, sq_ref):
    y = (x_ref[...] - m_ref[...]) / den_ref[...] * g_ref[...] + b_ref[...]
    y_ref[...] = y
    sq_ref[...] = jnp.sum(y * y, axis=1, keepdims=True)


def _bn_apply(x, m, den, g, b, br):
    n, f = x.shape
    small = [pl.BlockSpec((1, f), lambda i: (0, 0))] * 4
    return pl.pallas_call(
        _bn_apply_kernel,
        grid=(n // br,),
        in_specs=[pl.BlockSpec((br, f), lambda i: (i, 0))] + small,
        out_specs=[pl.BlockSpec((br, f), lambda i: (i, 0)),
                   pl.BlockSpec((br, 1), lambda i: (i, 0))],
        out_shape=[jax.ShapeDtypeStruct((n, f), jnp.float32),
                   jax.ShapeDtypeStruct((n, 1), jnp.float32)],
    )(x, m, den, g, b)


def _knn_kernel(lo_ref, hi_ref, xp_ref, xT_ref, sqr_ref, sqc_ref, br_ref,
                bc_ref, idxT_ref, *, bk):
    i = pl.program_id(0)
    row0 = i * bk
    xrTb = xT_ref[:, pl.ds(row0, bk)].astype(jnp.bfloat16)
    sqr = sqr_ref[:, pl.ds(row0, bk)]
    brow = br_ref[:, pl.ds(row0, bk)]
    c0 = lo_ref[i] // CW
    c1 = (hi_ref[i] + CW - 1) // CW
    sub = jax.lax.broadcasted_iota(jnp.int32, (CW + 32, 1), 0)
    sub_c = jax.lax.broadcasted_iota(jnp.int32, (CW, 1), 0)
    sub32 = jax.lax.broadcasted_iota(jnp.int32, (32, 1), 0)
    rowid = jax.lax.broadcasted_iota(jnp.int32, (1, bk), 1) + row0
    imax = jnp.int32(2 ** 30)
    pad_d = jnp.full((32 - K, bk), jnp.inf, jnp.float32)
    pad_i = jnp.zeros((32 - K, bk), jnp.int32)

    def body(c, carry):
        best_d, best_i = carry
        base = c * CW
        xcb = xp_ref[pl.ds(base, CW), :].astype(jnp.bfloat16)
        d = sqc_ref[pl.ds(base, CW), :] + sqr - 2.0 * jax.lax.dot_general(
            xcb, xrTb, (((1,), (0,)), ((), ())),
            preferred_element_type=jnp.float32)
        gcol = sub_c + base
        invalid = (bc_ref[pl.ds(base, CW), :] != brow) | (gcol == rowid)
        cat = jnp.concatenate([best_d, jnp.where(invalid, BIG, d)], axis=0)
        vals, poss = [], []
        for _ in range(K):
            m = jnp.min(cat, axis=0, keepdims=True)
            pos = jnp.min(jnp.where(cat == m, sub, imax), axis=0,
                          keepdims=True)
            vals.append(m)
            poss.append(pos)
            cat = jnp.where(sub == pos, jnp.inf, cat)
        nis = []
        for pos in poss:
            old = jnp.min(jnp.where(sub32 == pos, best_i, imax), axis=0,
                          keepdims=True)
            nis.append(jnp.where(pos < 32, old, base + pos - 32))
        new_d = jnp.concatenate(vals + [pad_d], axis=0)
        new_i = jnp.concatenate(nis + [pad_i], axis=0)
        return new_d, new_i

    best = (jnp.full((32, bk), jnp.inf, jnp.float32),
            jnp.zeros((32, bk), jnp.int32))
    _, best_i = jax.lax.fori_loop(c0, c1, body, best)
    idxT_ref[...] = best_i[:K, :]


def _knn(xp, xT_p, sqr_p, sqc_p, br_p, bc_p, lo, hi, bk):
    f = xp.shape[1]
    kern = lambda *a: _knn_kernel(*a, bk=bk)
    grid_spec = pltpu.PrefetchScalarGridSpec(
        num_scalar_prefetch=2,
        grid=(NPAD // bk,),
        in_specs=[pl.BlockSpec((NPAD, f), lambda i, *_: (0, 0)),
                  pl.BlockSpec((f, NPAD), lambda i, *_: (0, 0)),
                  pl.BlockSpec((1, NPAD), lambda i, *_: (0, 0)),
                  pl.BlockSpec((NPAD, 1), lambda i, *_: (0, 0)),
                  pl.BlockSpec((1, NPAD), lambda i, *_: (0, 0)),
                  pl.BlockSpec((NPAD, 1), lambda i, *_: (0, 0))],
        out_specs=pl.BlockSpec((K, bk), lambda i, *_: (0, i)),
    )
    idxT = pl.pallas_call(
        kern,
        grid_spec=grid_spec,
        out_shape=jax.ShapeDtypeStruct((K, NPAD), jnp.int32),
    )(lo, hi, xp, xT_p, sqr_p, sqc_p, br_p, bc_p)
    return idxT.T[:N]


def _sc_gather(table, idx, chunk=640):
    d_real = table.shape[1]
    table = jnp.pad(table, ((0, 0), (0, 128 - d_real)))
    e_pad = idx.shape[0]
    d = table.shape[1]
    nw = 32
    b_per_w = e_pad // nw
    nch = b_per_w // chunk

    @functools.partial(
        pl.kernel,
        mesh=plsc.VectorSubcoreMesh(core_axis_name="c", subcore_axis_name="s"),
        out_type=jax.ShapeDtypeStruct((e_pad, d), jnp.float32),
        scratch_types=[
            pltpu.VMEM((chunk,), jnp.int32),
            pltpu.VMEM((chunk, d), jnp.float32),
            pltpu.SemaphoreType.DMA,
        ],
    )
    def gk(table_hbm, idx_hbm, out_hbm, idx_v, rows_v, sem):
        wid = lax.axis_index("s") * 2 + lax.axis_index("c")
        base = wid * b_per_w

        def body(j, _):
            off = base + j * chunk
            pltpu.sync_copy(idx_hbm.at[pl.ds(off, chunk)], idx_v)
            pltpu.async_copy(table_hbm.at[idx_v], rows_v, sem).wait()
            pltpu.sync_copy(rows_v, out_hbm.at[pl.ds(off, chunk)])
            return 0

        lax.fori_loop(0, nch, body, 0)

    return gk(table, idx)[:, :d_real]


def _xa_kernel(x_ref, wa_ref, a_ref):
    a_ref[...] = _bdot(x_ref[...], wa_ref[...])


def _xa(x, wa, br):
    n, f = x.shape
    h = wa.shape[1]
    return pl.pallas_call(
        _xa_kernel,
        grid=(n // br,),
        in_specs=[pl.BlockSpec((br, f), lambda i: (i, 0)),
                  pl.BlockSpec((f, h), lambda i: (0, 0))],
        out_specs=pl.BlockSpec((br, h), lambda i: (i, 0)),
        out_shape=jax.ShapeDtypeStruct((n, h), jnp.float32),
    )(x, wa)


def _l1_kernel(xi_ref, xj_ref, a_ref, wb_ref, b_ref, pre_ref, s_ref, q_ref,
               *, bn, f, h):
    i = pl.program_id(0)

    @pl.when(i == 0)
    def _():
        s_ref[...] = jnp.zeros_like(s_ref)
        q_ref[...] = jnp.zeros_like(q_ref)

    t = xj_ref[...].reshape(bn, K, f) - xi_ref[...][:, None, :]
    td = _bdot(t.reshape(bn * K, f), wb_ref[...]).reshape(bn, K, h)
    pre = ((a_ref[...][:, None, :] + td) + b_ref[...]).reshape(bn * K, h)
    pre_ref[...] = pre
    s_ref[...] += jnp.sum(pre, axis=0, keepdims=True)
    q_ref[...] += jnp.sum(pre * pre, axis=0, keepdims=True)


def _l1(xi, xj, a, wb, b, bn):
    n, h = a.shape
    f = xi.shape[1]
    kern = lambda *r: _l1_kernel(*r, bn=bn, f=f, h=h)
    return pl.pallas_call(
        kern,
        grid=(n // bn,),
        in_specs=[pl.BlockSpec((bn, f), lambda i: (i, 0)),
                  pl.BlockSpec((bn * K, f), lambda i: (i, 0)),
                  pl.BlockSpec((bn, h), lambda i: (i, 0)),
                  pl.BlockSpec((f, h), lambda i: (0, 0)),
                  pl.BlockSpec((1, h), lambda i: (0, 0))],
        out_specs=[pl.BlockSpec((bn * K, h), lambda i: (i, 0)),
                   pl.BlockSpec((1, h), lambda i: (0, 0)),
                   pl.BlockSpec((1, h), lambda i: (0, 0))],
        out_shape=[jax.ShapeDtypeStruct((n * K, h), jnp.float32),
                   jax.ShapeDtypeStruct((1, h), jnp.float32),
                   jax.ShapeDtypeStruct((1, h), jnp.float32)],
    )(xi, xj, a, wb, b)


def _l2_kernel(pre_ref, m_ref, den_ref, g_ref, be_ref, w_ref, b_ref,
               o_ref, s_ref, q_ref):
    i = pl.program_id(0)

    @pl.when(i == 0)
    def _():
        s_ref[...] = jnp.zeros_like(s_ref)
        q_ref[...] = jnp.zeros_like(q_ref)

    hact = jnp.maximum(
        (pre_ref[...] - m_ref[...]) / den_ref[...] * g_ref[...] + be_ref[...],
        0.0)
    o = _bdot(hact, w_ref[...]) + b_ref[...]
    o_ref[...] = o
    s_ref[...] += jnp.sum(o, axis=0, keepdims=True)
    q_ref[...] += jnp.sum(o * o, axis=0, keepdims=True)


def _l2(pre, m, den, g, be, w, b, bblk):
    e, h = pre.shape
    ho = w.shape[1]
    small = [pl.BlockSpec((1, h), lambda i: (0, 0))] * 4
    return pl.pallas_call(
        _l2_kernel,
        grid=(e // bblk,),
        in_specs=[pl.BlockSpec((bblk, h), lambda i: (i, 0))] + small
        + [pl.BlockSpec((h, ho), lambda i: (0, 0)),
           pl.BlockSpec((1, ho), lambda i: (0, 0))],
        out_specs=[pl.BlockSpec((bblk, ho), lambda i: (i, 0)),
                   pl.BlockSpec((1, ho), lambda i: (0, 0)),
                   pl.BlockSpec((1, ho), lambda i: (0, 0))],
        out_shape=[jax.ShapeDtypeStruct((e, ho), jnp.float32),
                   jax.ShapeDtypeStruct((1, ho), jnp.float32),
                   jax.ShapeDtypeStruct((1, ho), jnp.float32)],
    )(pre, m, den, g, be, w, b)


def _l3_kernel(pre_ref, m_ref, den_ref, g_ref, be_ref, o_ref, sq_ref,
               *, bn, h):
    hact = jnp.maximum(
        (pre_ref[...] - m_ref[...]) / den_ref[...] * g_ref[...] + be_ref[...],
        0.0)
    xo = jnp.max(hact.reshape(bn, K, h), axis=1)
    o_ref[...] = xo
    sq_ref[...] = jnp.sum(xo * xo, axis=1, keepdims=True)


def _l3(pre, m, den, g, be, bn):
    e, h = pre.shape
    n = e // K
    small = [pl.BlockSpec((1, h), lambda i: (0, 0))] * 4
    kern = lambda *r: _l3_kernel(*r, bn=bn, h=h)
    return pl.pallas_call(
        kern,
        grid=(n // bn,),
        in_specs=[pl.BlockSpec((bn * K, h), lambda i: (i, 0))] + small,
        out_specs=[pl.BlockSpec((bn, h), lambda i: (i, 0)),
                   pl.BlockSpec((bn, 1), lambda i: (i, 0))],
        out_shape=[jax.ShapeDtypeStruct((n, h), jnp.float32),
                   jax.ShapeDtypeStruct((n, 1), jnp.float32)],
    )(pre, m, den, g, be)


def _stats(s, q, cnt, g, be):
    m = s / cnt
    v = q / cnt - m * m
    den = jnp.sqrt(v + 1e-5)
    return m, den, g.reshape(1, -1), be.reshape(1, -1)


def _edge_conv(x_in, xj, w1, b1, g1, be1, w2, b2, g2, be2, bn):
    n, fin = x_in.shape
    a = _xa(x_in, w1[:fin], 2000)
    pre1, s1, q1 = _l1(x_in, xj, a, w1[fin:], b1.reshape(1, -1), bn)
    pre2, s2, q2 = _l2(pre1, *_stats(s1, q1, n * K, g1, be1),
                       w2, b2.reshape(1, -1), bn * K)
    return _l3(pre2, *_stats(s2, q2, n * K, g2, be2), bn)


def _cls1_kernel(x1_ref, x2_ref, wt_ref, wb_ref, b_ref, o_ref, s_ref, q_ref):
    i = pl.program_id(0)

    @pl.when(i == 0)
    def _():
        s_ref[...] = jnp.zeros_like(s_ref)
        q_ref[...] = jnp.zeros_like(q_ref)

    o = _bdot(x1_ref[...], wt_ref[...]) + _bdot(x2_ref[...], wb_ref[...]) \
        + b_ref[...]
    o_ref[...] = o
    s_ref[...] += jnp.sum(o, axis=0, keepdims=True)
    q_ref[...] += jnp.sum(o * o, axis=0, keepdims=True)


def _cls1(x1, x2, wt, wb, b, br):
    n, h1 = x1.shape
    h2 = x2.shape[1]
    ho = wt.shape[1]
    return pl.pallas_call(
        _cls1_kernel,
        grid=(n // br,),
        in_specs=[pl.BlockSpec((br, h1), lambda i: (i, 0)),
                  pl.BlockSpec((br, h2), lambda i: (i, 0)),
                  pl.BlockSpec((h1, ho), lambda i: (0, 0)),
                  pl.BlockSpec((h2, ho), lambda i: (0, 0)),
                  pl.BlockSpec((1, ho), lambda i: (0, 0))],
        out_specs=[pl.BlockSpec((br, ho), lambda i: (i, 0)),
                   pl.BlockSpec((1, ho), lambda i: (0, 0)),
                   pl.BlockSpec((1, ho), lambda i: (0, 0))],
        out_shape=[jax.ShapeDtypeStruct((n, ho), jnp.float32),
                   jax.ShapeDtypeStruct((1, ho), jnp.float32),
                   jax.ShapeDtypeStruct((1, ho), jnp.float32)],
    )(x1, x2, wt, wb, b)


def _cls2_kernel(pre_ref, m_ref, den_ref, g_ref, be_ref, wT_ref, b_ref, o_ref):
    hact = jnp.maximum(
        (pre_ref[...] - m_ref[...]) / den_ref[...] * g_ref[...] + be_ref[...],
        0.0)
    hb = hact.astype(jnp.bfloat16).astype(jnp.float32)
    wb = wT_ref[...].astype(jnp.bfloat16).astype(jnp.float32)
    o_ref[...] = jnp.sum(hb * wb, axis=1, keepdims=True) + b_ref[...]


def _cls2(pre, m, den, g, be, wT, b, br):
    n, h = pre.shape
    small = [pl.BlockSpec((1, h), lambda i: (0, 0))] * 5
    return pl.pallas_call(
        _cls2_kernel,
        grid=(n // br,),
        in_specs=[pl.BlockSpec((br, h), lambda i: (i, 0))] + small
        + [pl.BlockSpec((1, 1), lambda i: (0, 0))],
        out_specs=pl.BlockSpec((br, 1), lambda i: (i, 0)),
        out_shape=jax.ShapeDtypeStruct((n, 1), jnp.float32),
    )(pre, m, den, g, be, wT, b)


def kernel(x, batch, in_gamma, in_beta, c1_w1, c1_b1, c1_g1, c1_be1,
           c1_w2, c1_b2, c1_g2, c1_be2, c2_w1, c2_b1, c2_g1, c2_be1,
           c2_w2, c2_b2, c2_g2, c2_be2, cl_w1, cl_b1, cl_g, cl_be,
           cl_w2, cl_b2):
    n = x.shape[0]
    bk = 1024
    batch = batch.astype(jnp.int32)
    br_p = jnp.pad(batch.reshape(1, n), ((0, 0), (0, NPAD - n)),
                   constant_values=-2)
    bc_p = jnp.pad(batch.reshape(n, 1), ((0, NPAD - n), (0, 0)),
                   constant_values=-1)
    batch_hi = jnp.pad(batch, (0, NPAD - n), constant_values=7)
    lo = jnp.searchsorted(batch, batch_hi[0::bk], side='left').astype(jnp.int32)
    hi = jnp.searchsorted(batch, batch_hi[bk - 1::bk], side='right').astype(jnp.int32)

    padc = lambda a: jnp.pad(a, ((0, 0), (0, NPAD - n)))
    padr = lambda a: jnp.pad(a, ((0, NPAD - n), (0, 0)))

    m0 = jnp.mean(x, axis=0, keepdims=True)
    den0 = jnp.sqrt(jnp.var(x, axis=0, keepdims=True) + 1e-5)
    x0, sq0 = _bn_apply(x, m0, den0, in_gamma.reshape(1, -1),
                        in_beta.reshape(1, -1), 2000)

    idx1 = _knn(padr(x0), padc(x0.T), padc(sq0.reshape(1, n)), padr(sq0),
                br_p, bc_p, lo, hi, bk)
    epad = 204800
    xj1 = _sc_gather(x0, jnp.pad(idx1.reshape(-1), (0, epad - n * K)))[:n * K]
    x1, sq1 = _edge_conv(x0, xj1, c1_w1, c1_b1, c1_g1, c1_be1,
                         c1_w2, c1_b2, c1_g2, c1_be2, 200)

    idx2 = _knn(padr(x1), padc(x1.T), padc(sq1.reshape(1, n)), padr(sq1),
                br_p, bc_p, lo, hi, bk)
    xj2 = _sc_gather(x1, jnp.pad(idx2.reshape(-1), (0, epad - n * K)))[:n * K]

    x2, _ = _edge_conv(x1, xj2, c2_w1, c2_b1, c2_g1, c2_be1,
                       c2_w2, c2_b2, c2_g2, c2_be2, 200)

    pre, s, q = _cls1(x1, x2, cl_w1[:x1.shape[1]], cl_w1[x1.shape[1]:],
                      cl_b1.reshape(1, -1), 2000)
    return _cls2(pre, *_stats(s, q, n, cl_g, cl_be),
                 cl_w2.reshape(1, -1), cl_b2.reshape(1, 1), 2000)


# speedup vs baseline: 1.0656x; 1.0656x over previous
import functools

import jax
import jax.numpy as jnp
from jax import lax
from jax.experimental import pallas as pl
from jax.experimental.pallas import tpu as pltpu
from jax.experimental.pallas import tpu_sc as plsc

N = 10000
K = 20
BIG = 1e30
CW = 512
NPAD = 10240


def _bdot(a, b):
    return jax.lax.dot_general(
        a.astype(jnp.bfloat16), b.astype(jnp.bfloat16),
        (((1,), (0,)), ((), ())), preferred_element_type=jnp.float32)


def _bn_apply_kernel(x_ref, m_ref, den_ref, g_ref, b_ref, y_ref, sq_ref):
    y = (x_ref[...] - m_ref[...]) / den_ref[...] * g_ref[...] + b_ref[...]
    y_ref[...] = y
    sq_ref[...] = jnp.sum(y * y, axis=1, keepdims=True)


def _bn_apply(x, m, den, g, b, br):
    n, f = x.shape
    small = [pl.BlockSpec((1, f), lambda i: (0, 0))] * 4
    return pl.pallas_call(
        _bn_apply_kernel,
        grid=(n // br,),
        in_specs=[pl.BlockSpec((br, f), lambda i: (i, 0))] + small,
        out_specs=[pl.BlockSpec((br, f), lambda i: (i, 0)),
                   pl.BlockSpec((br, 1), lambda i: (i, 0))],
        out_shape=[jax.ShapeDtypeStruct((n, f), jnp.float32),
                   jax.ShapeDtypeStruct((n, 1), jnp.float32)],
    )(x, m, den, g, b)


def _knn_kernel(lo_ref, hi_ref, xp_ref, xT_ref, sqr_ref, sqc_ref, br_ref,
                bc_ref, idxT_ref, *, bk):
    i = pl.program_id(0)
    row0 = i * bk
    xrTb = xT_ref[:, pl.ds(row0, bk)].astype(jnp.bfloat16)
    sqr = sqr_ref[:, pl.ds(row0, bk)]
    brow = br_ref[:, pl.ds(row0, bk)]
    c0 = lo_ref[i] // CW
    c1 = (hi_ref[i] + CW - 1) // CW
    sub = jax.lax.broadcasted_iota(jnp.int32, (CW + 32, 1), 0)
    sub_c = jax.lax.broadcasted_iota(jnp.int32, (CW, 1), 0)
    sub32 = jax.lax.broadcasted_iota(jnp.int32, (32, 1), 0)
    rowid = jax.lax.broadcasted_iota(jnp.int32, (1, bk), 1) + row0
    imax = jnp.int32(2 ** 30)
    pad_d = jnp.full((32 - K, bk), jnp.inf, jnp.float32)
    pad_i = jnp.zeros((32 - K, bk), jnp.int32)

    def body(c, carry):
        best_d, best_i = carry
        base = c * CW
        xcb = xp_ref[pl.ds(base, CW), :].astype(jnp.bfloat16)
        d = sqc_ref[pl.ds(base, CW), :] + sqr - 2.0 * jax.lax.dot_general(
            xcb, xrTb, (((1,), (0,)), ((), ())),
            preferred_element_type=jnp.float32)
        gcol = sub_c + base
        invalid = (bc_ref[pl.ds(base, CW), :] != brow) | (gcol == rowid)
        cat = jnp.concatenate([best_d, jnp.where(invalid, BIG, d)], axis=0)
        vals, poss = [], []
        for _ in range(K):
            m = jnp.min(cat, axis=0, keepdims=True)
            pos = jnp.min(jnp.where(cat == m, sub, imax), axis=0,
                          keepdims=True)
            vals.append(m)
            poss.append(pos)
            cat = jnp.where(sub == pos, jnp.inf, cat)
        nis = []
        for pos in poss:
            old = jnp.min(jnp.where(sub32 == pos, best_i, imax), axis=0,
                          keepdims=True)
            nis.append(jnp.where(pos < 32, old, base + pos - 32))
        new_d = jnp.concatenate(vals + [pad_d], axis=0)
        new_i = jnp.concatenate(nis + [pad_i], axis=0)
        return new_d, new_i

    best = (jnp.full((32, bk), jnp.inf, jnp.float32),
            jnp.zeros((32, bk), jnp.int32))
    _, best_i = jax.lax.fori_loop(c0, c1, body, best)
    idxT_ref[...] = best_i[:K, :]


def _knn(xp, xT_p, sqr_p, sqc_p, br_p, bc_p, lo, hi, bk):
    f = xp.shape[1]
    kern = lambda *a: _knn_kernel(*a, bk=bk)
    grid_spec = pltpu.PrefetchScalarGridSpec(
        num_scalar_prefetch=2,
        grid=(NPAD // bk,),
        in_specs=[pl.BlockSpec((NPAD, f), lambda i, *_: (0, 0)),
                  pl.BlockSpec((f, NPAD), lambda i, *_: (0, 0)),
                  pl.BlockSpec((1, NPAD), lambda i, *_: (0, 0)),
                  pl.BlockSpec((NPAD, 1), lambda i, *_: (0, 0)),
                  pl.BlockSpec((1, NPAD), lambda i, *_: (0, 0)),
                  pl.BlockSpec((NPAD, 1), lambda i, *_: (0, 0))],
        out_specs=pl.BlockSpec((K, bk), lambda i, *_: (0, i)),
    )
    idxT = pl.pallas_call(
        kern,
        grid_spec=grid_spec,
        out_shape=jax.ShapeDtypeStruct((K, NPAD), jnp.int32),
    )(lo, hi, xp, xT_p, sqr_p, sqc_p, br_p, bc_p)
    return idxT.T[:N]


def _sc_gather(table, idx, chunk=640):
    d_real = table.shape[1]
    table = jnp.pad(table, ((0, 0), (0, 128 - d_real)))
    e_pad = idx.shape[0]
    d = table.shape[1]
    nw = 32
    b_per_w = e_pad // nw
    nch = b_per_w // chunk

    @functools.partial(
        pl.kernel,
        mesh=plsc.VectorSubcoreMesh(core_axis_name="c", subcore_axis_name="s"),
        out_type=jax.ShapeDtypeStruct((e_pad, d), jnp.float32),
        scratch_types=[
            pltpu.VMEM((chunk,), jnp.int32),
            pltpu.VMEM((chunk, d), jnp.float32),
            pltpu.SemaphoreType.DMA,
        ],
    )
    def gk(table_hbm, idx_hbm, out_hbm, idx_v, rows_v, sem):
        wid = lax.axis_index("s") * 2 + lax.axis_index("c")
        base = wid * b_per_w

        def body(j, _):
            off = base + j * chunk
            pltpu.sync_copy(idx_hbm.at[pl.ds(off, chunk)], idx_v)
            pltpu.async_copy(table_hbm.at[idx_v], rows_v, sem).wait()
            pltpu.sync_copy(rows_v, out_hbm.at[pl.ds(off, chunk)])
            return 0

        lax.fori_loop(0, nch, body, 0)

    return gk(table, idx)[:, :d_real]


def _xa_kernel(x_ref, wa_ref, a_ref):
    a_ref[...] = _bdot(x_ref[...], wa_ref[...])


def _xa(x, wa, br):
    n, f = x.shape
    h = wa.shape[1]
    return pl.pallas_call(
        _xa_kernel,
        grid=(n // br,),
        in_specs=[pl.BlockSpec((br, f), lambda i: (i, 0)),
                  pl.BlockSpec((f, h), lambda i: (0, 0))],
        out_specs=pl.BlockSpec((br, h), lambda i: (i, 0)),
        out_shape=jax.ShapeDtypeStruct((n, h), jnp.float32),
    )(x, wa)


def _l1_kernel(xi_ref, xj_ref, a_ref, wb_ref, b_ref, pre_ref, s_ref, q_ref,
               *, bn, f, h):
    i = pl.program_id(0)

    @pl.when(i == 0)
    def _():
        s_ref[...] = jnp.zeros_like(s_ref)
        q_ref[...] = jnp.zeros_like(q_ref)

    t = xj_ref[...].reshape(bn, K, f) - xi_ref[...][:, None, :]
    td = _bdot(t.reshape(bn * K, f), wb_ref[...]).reshape(bn, K, h)
    pre = ((a_ref[...][:, None, :] + td) + b_ref[...]).reshape(bn * K, h)
    pre_ref[...] = pre
    s_ref[...] += jnp.sum(pre, axis=0, keepdims=True)
    q_ref[...] += jnp.sum(pre * pre, axis=0, keepdims=True)


def _l1(xi, xj, a, wb, b, bn):
    n, h = a.shape
    f = xi.shape[1]
    kern = lambda *r: _l1_kernel(*r, bn=bn, f=f, h=h)
    return pl.pallas_call(
        kern,
        grid=(n // bn,),
        in_specs=[pl.BlockSpec((bn, f), lambda i: (i, 0)),
                  pl.BlockSpec((bn * K, f), lambda i: (i, 0)),
                  pl.BlockSpec((bn, h), lambda i: (i, 0)),
                  pl.BlockSpec((f, h), lambda i: (0, 0)),
                  pl.BlockSpec((1, h), lambda i: (0, 0))],
        out_specs=[pl.BlockSpec((bn * K, h), lambda i: (i, 0)),
                   pl.BlockSpec((1, h), lambda i: (0, 0)),
                   pl.BlockSpec((1, h), lambda i: (0, 0))],
        out_shape=[jax.ShapeDtypeStruct((n * K, h), jnp.float32),
                   jax.ShapeDtypeStruct((1, h), jnp.float32),
                   jax.ShapeDtypeStruct((1, h), jnp.float32)],
    )(xi, xj, a, wb, b)


def _l2_kernel(pre_ref, m_ref, den_ref, g_ref, be_ref, w_ref, b_ref,
               o_ref, s_ref, q_ref):
    i = pl.program_id(0)

    @pl.when(i == 0)
    def _():
        s_ref[...] = jnp.zeros_like(s_ref)
        q_ref[...] = jnp.zeros_like(q_ref)

    hact = jnp.maximum(
        (pre_ref[...] - m_ref[...]) / den_ref[...] * g_ref[...] + be_ref[...],
        0.0)
    o = _bdot(hact, w_ref[...]) + b_ref[...]
    o_ref[...] = o
    s_ref[...] += jnp.sum(o, axis=0, keepdims=True)
    q_ref[...] += jnp.sum(o * o, axis=0, keepdims=True)


def _l2(pre, m, den, g, be, w, b, bblk):
    e, h = pre.shape
    ho = w.shape[1]
    small = [pl.BlockSpec((1, h), lambda i: (0, 0))] * 4
    return pl.pallas_call(
        _l2_kernel,
        grid=(e // bblk,),
        in_specs=[pl.BlockSpec((bblk, h), lambda i: (i, 0))] + small
        + [pl.BlockSpec((h, ho), lambda i: (0, 0)),
           pl.BlockSpec((1, ho), lambda i: (0, 0))],
        out_specs=[pl.BlockSpec((bblk, ho), lambda i: (i, 0)),
                   pl.BlockSpec((1, ho), lambda i: (0, 0)),
                   pl.BlockSpec((1, ho), lambda i: (0, 0))],
        out_shape=[jax.ShapeDtypeStruct((e, ho), jnp.float32),
                   jax.ShapeDtypeStruct((1, ho), jnp.float32),
                   jax.ShapeDtypeStruct((1, ho), jnp.float32)],
    )(pre, m, den, g, be, w, b)


def _l3_kernel(pre_ref, m_ref, den_ref, g_ref, be_ref, o_ref, sq_ref,
               *, bn, h):
    hact = jnp.maximum(
        (pre_ref[...] - m_ref[...]) / den_ref[...] * g_ref[...] + be_ref[...],
        0.0)
    xo = jnp.max(hact.reshape(bn, K, h), axis=1)
    o_ref[...] = xo
    sq_ref[...] = jnp.sum(xo * xo, axis=1, keepdims=True)


def _l3(pre, m, den, g, be, bn):
    e, h = pre.shape
    n = e // K
    small = [pl.BlockSpec((1, h), lambda i: (0, 0))] * 4
    kern = lambda *r: _l3_kernel(*r, bn=bn, h=h)
    return pl.pallas_call(
        kern,
        grid=(n // bn,),
        in_specs=[pl.BlockSpec((bn * K, h), lambda i: (i, 0))] + small,
        out_specs=[pl.BlockSpec((bn, h), lambda i: (i, 0)),
                   pl.BlockSpec((bn, 1), lambda i: (i, 0))],
        out_shape=[jax.ShapeDtypeStruct((n, h), jnp.float32),
                   jax.ShapeDtypeStruct((n, 1), jnp.float32)],
    )(pre, m, den, g, be)


def _stats(s, q, cnt, g, be):
    m = s / cnt
    v = q / cnt - m * m
    den = jnp.sqrt(v + 1e-5)
    return m, den, g.reshape(1, -1), be.reshape(1, -1)


def _edge_conv(x_in, xj, w1, b1, g1, be1, w2, b2, g2, be2, bn):
    n, fin = x_in.shape
    a = _xa(x_in, w1[:fin], 2000)
    pre1, s1, q1 = _l1(x_in, xj, a, w1[fin:], b1.reshape(1, -1), bn)
    pre2, s2, q2 = _l2(pre1, *_stats(s1, q1, n * K, g1, be1),
                       w2, b2.reshape(1, -1), bn * K)
    return _l3(pre2, *_stats(s2, q2, n * K, g2, be2), bn)


def _cls1_kernel(x1_ref, x2_ref, wt_ref, wb_ref, b_ref, o_ref, s_ref, q_ref):
    i = pl.program_id(0)

    @pl.when(i == 0)
    def _():
        s_ref[...] = jnp.zeros_like(s_ref)
        q_ref[...] = jnp.zeros_like(q_ref)

    o = _bdot(x1_ref[...], wt_ref[...]) + _bdot(x2_ref[...], wb_ref[...]) \
        + b_ref[...]
    o_ref[...] = o
    s_ref[...] += jnp.sum(o, axis=0, keepdims=True)
    q_ref[...] += jnp.sum(o * o, axis=0, keepdims=True)


def _cls1(x1, x2, wt, wb, b, br):
    n, h1 = x1.shape
    h2 = x2.shape[1]
    ho = wt.shape[1]
    return pl.pallas_call(
        _cls1_kernel,
        grid=(n // br,),
        in_specs=[pl.BlockSpec((br, h1), lambda i: (i, 0)),
                  pl.BlockSpec((br, h2), lambda i: (i, 0)),
                  pl.BlockSpec((h1, ho), lambda i: (0, 0)),
                  pl.BlockSpec((h2, ho), lambda i: (0, 0)),
                  pl.BlockSpec((1, ho), lambda i: (0, 0))],
        out_specs=[pl.BlockSpec((br, ho), lambda i: (i, 0)),
                   pl.BlockSpec((1, ho), lambda i: (0, 0)),
                   pl.BlockSpec((1, ho), lambda i: (0, 0))],
        out_shape=[jax.ShapeDtypeStruct((n, ho), jnp.float32),
                   jax.ShapeDtypeStruct((1, ho), jnp.float32),
                   jax.ShapeDtypeStruct((1, ho), jnp.float32)],
    )(x1, x2, wt, wb, b)


def _cls2_kernel(pre_ref, m_ref, den_ref, g_ref, be_ref, wT_ref, b_ref, o_ref):
    hact = jnp.maximum(
        (pre_ref[...] - m_ref[...]) / den_ref[...] * g_ref[...] + be_ref[...],
        0.0)
    hb = hact.astype(jnp.bfloat16).astype(jnp.float32)
    wb = wT_ref[...].astype(jnp.bfloat16).astype(jnp.float32)
    o_ref[...] = jnp.sum(hb * wb, axis=1, keepdims=True) + b_ref[...]


def _cls2(pre, m, den, g, be, wT, b, br):
    n, h = pre.shape
    small = [pl.BlockSpec((1, h), lambda i: (0, 0))] * 5
    return pl.pallas_call(
        _cls2_kernel,
        grid=(n // br,),
        in_specs=[pl.BlockSpec((br, h), lambda i: (i, 0))] + small
        + [pl.BlockSpec((1, 1), lambda i: (0, 0))],
        out_specs=pl.BlockSpec((br, 1), lambda i: (i, 0)),
        out_shape=jax.ShapeDtypeStruct((n, 1), jnp.float32),
    )(pre, m, den, g, be, wT, b)


def kernel(x, batch, in_gamma, in_beta, c1_w1, c1_b1, c1_g1, c1_be1,
           c1_w2, c1_b2, c1_g2, c1_be2, c2_w1, c2_b1, c2_g1, c2_be1,
           c2_w2, c2_b2, c2_g2, c2_be2, cl_w1, cl_b1, cl_g, cl_be,
           cl_w2, cl_b2):
    n = x.shape[0]
    bk = 512
    batch = batch.astype(jnp.int32)
    br_p = jnp.pad(batch.reshape(1, n), ((0, 0), (0, NPAD - n)),
                   constant_values=-2)
    bc_p = jnp.pad(batch.reshape(n, 1), ((0, NPAD - n), (0, 0)),
                   constant_values=-1)
    batch_hi = jnp.pad(batch, (0, NPAD - n), constant_values=7)
    lo = jnp.searchsorted(batch, batch_hi[0::bk], side='left').astype(jnp.int32)
    hi = jnp.searchsorted(batch, batch_hi[bk - 1::bk], side='right').astype(jnp.int32)

    padc = lambda a: jnp.pad(a, ((0, 0), (0, NPAD - n)))
    padr = lambda a: jnp.pad(a, ((0, NPAD - n), (0, 0)))

    m0 = jnp.mean(x, axis=0, keepdims=True)
    den0 = jnp.sqrt(jnp.var(x, axis=0, keepdims=True) + 1e-5)
    x0, sq0 = _bn_apply(x, m0, den0, in_gamma.reshape(1, -1),
                        in_beta.reshape(1, -1), 2000)

    idx1 = _knn(padr(x0), padc(x0.T), padc(sq0.reshape(1, n)), padr(sq0),
                br_p, bc_p, lo, hi, bk)
    epad = 204800
    xj1 = _sc_gather(x0, jnp.pad(idx1.reshape(-1), (0, epad - n * K)))[:n * K]
    x1, sq1 = _edge_conv(x0, xj1, c1_w1, c1_b1, c1_g1, c1_be1,
                         c1_w2, c1_b2, c1_g2, c1_be2, 200)

    idx2 = _knn(padr(x1), padc(x1.T), padc(sq1.reshape(1, n)), padr(sq1),
                br_p, bc_p, lo, hi, bk)
    xj2 = _sc_gather(x1, jnp.pad(idx2.reshape(-1), (0, epad - n * K)))[:n * K]

    x2, _ = _edge_conv(x1, xj2, c2_w1, c2_b1, c2_g1, c2_be1,
                       c2_w2, c2_b2, c2_g2, c2_be2, 200)

    pre, s, q = _cls1(x1, x2, cl_w1[:x1.shape[1]], cl_w1[x1.shape[1]:],
                      cl_b1.reshape(1, -1), 2000)
    return _cls2(pre, *_stats(s, q, n, cl_g, cl_be),
                 cl_w2.reshape(1, -1), cl_b2.reshape(1, 1), 2000)


# speedup vs baseline: 1.0892x; 1.0221x over previous
import functools

import jax
import jax.numpy as jnp
from jax import lax
from jax.experimental import pallas as pl
from jax.experimental.pallas import tpu as pltpu
from jax.experimental.pallas import tpu_sc as plsc

N = 10000
K = 20
BIG = 1e30
CW = 256
NPAD = 10240


def _bdot(a, b):
    return jax.lax.dot_general(
        a.astype(jnp.bfloat16), b.astype(jnp.bfloat16),
        (((1,), (0,)), ((), ())), preferred_element_type=jnp.float32)


def _bn_apply_kernel(x_ref, m_ref, den_ref, g_ref, b_ref, y_ref, sq_ref):
    y = (x_ref[...] - m_ref[...]) / den_ref[...] * g_ref[...] + b_ref[...]
    y_ref[...] = y
    sq_ref[...] = jnp.sum(y * y, axis=1, keepdims=True)


def _bn_apply(x, m, den, g, b, br):
    n, f = x.shape
    small = [pl.BlockSpec((1, f), lambda i: (0, 0))] * 4
    return pl.pallas_call(
        _bn_apply_kernel,
        grid=(n // br,),
        in_specs=[pl.BlockSpec((br, f), lambda i: (i, 0))] + small,
        out_specs=[pl.BlockSpec((br, f), lambda i: (i, 0)),
                   pl.BlockSpec((br, 1), lambda i: (i, 0))],
        out_shape=[jax.ShapeDtypeStruct((n, f), jnp.float32),
                   jax.ShapeDtypeStruct((n, 1), jnp.float32)],
    )(x, m, den, g, b)


def _knn_kernel(lo_ref, hi_ref, xp_ref, xT_ref, sqr_ref, sqc_ref, br_ref,
                bc_ref, idxT_ref, *, bk):
    i = pl.program_id(0)
    row0 = i * bk
    xrTb = xT_ref[:, pl.ds(row0, bk)].astype(jnp.bfloat16)
    sqr = sqr_ref[:, pl.ds(row0, bk)]
    brow = br_ref[:, pl.ds(row0, bk)]
    c0 = lo_ref[i] // CW
    c1 = (hi_ref[i] + CW - 1) // CW
    sub = jax.lax.broadcasted_iota(jnp.int32, (CW + 32, 1), 0)
    sub_c = jax.lax.broadcasted_iota(jnp.int32, (CW, 1), 0)
    sub32 = jax.lax.broadcasted_iota(jnp.int32, (32, 1), 0)
    rowid = jax.lax.broadcasted_iota(jnp.int32, (1, bk), 1) + row0
    imax = jnp.int32(2 ** 30)
    pad_d = jnp.full((32 - K, bk), jnp.inf, jnp.float32)
    pad_i = jnp.zeros((32 - K, bk), jnp.int32)

    def body(c, carry):
        best_d, best_i = carry
        base = c * CW
        xcb = xp_ref[pl.ds(base, CW), :].astype(jnp.bfloat16)
        d = sqc_ref[pl.ds(base, CW), :] + sqr - 2.0 * jax.lax.dot_general(
            xcb, xrTb, (((1,), (0,)), ((), ())),
            preferred_element_type=jnp.float32)
        gcol = sub_c + base
        invalid = (bc_ref[pl.ds(base, CW), :] != brow) | (gcol == rowid)
        cat = jnp.concatenate([best_d, jnp.where(invalid, BIG, d)], axis=0)
        vals, poss = [], []
        for _ in range(K):
            m = jnp.min(cat, axis=0, keepdims=True)
            pos = jnp.min(jnp.where(cat == m, sub, imax), axis=0,
                          keepdims=True)
            vals.append(m)
            poss.append(pos)
            cat = jnp.where(sub == pos, jnp.inf, cat)
        nis = []
        for pos in poss:
            old = jnp.min(jnp.where(sub32 == pos, best_i, imax), axis=0,
                          keepdims=True)
            nis.append(jnp.where(pos < 32, old, base + pos - 32))
        new_d = jnp.concatenate(vals + [pad_d], axis=0)
        new_i = jnp.concatenate(nis + [pad_i], axis=0)
        return new_d, new_i

    best = (jnp.full((32, bk), jnp.inf, jnp.float32),
            jnp.zeros((32, bk), jnp.int32))
    _, best_i = jax.lax.fori_loop(c0, c1, body, best)
    idxT_ref[...] = best_i[:K, :]


def _knn(xp, xT_p, sqr_p, sqc_p, br_p, bc_p, lo, hi, bk):
    f = xp.shape[1]
    kern = lambda *a: _knn_kernel(*a, bk=bk)
    grid_spec = pltpu.PrefetchScalarGridSpec(
        num_scalar_prefetch=2,
        grid=(NPAD // bk,),
        in_specs=[pl.BlockSpec((NPAD, f), lambda i, *_: (0, 0)),
                  pl.BlockSpec((f, NPAD), lambda i, *_: (0, 0)),
                  pl.BlockSpec((1, NPAD), lambda i, *_: (0, 0)),
                  pl.BlockSpec((NPAD, 1), lambda i, *_: (0, 0)),
                  pl.BlockSpec((1, NPAD), lambda i, *_: (0, 0)),
                  pl.BlockSpec((NPAD, 1), lambda i, *_: (0, 0))],
        out_specs=pl.BlockSpec((K, bk), lambda i, *_: (0, i)),
    )
    idxT = pl.pallas_call(
        kern,
        grid_spec=grid_spec,
        out_shape=jax.ShapeDtypeStruct((K, NPAD), jnp.int32),
    )(lo, hi, xp, xT_p, sqr_p, sqc_p, br_p, bc_p)
    return idxT.T[:N]


def _sc_gather(table, idx, chunk=800):
    d_real = table.shape[1]
    table = jnp.pad(table, ((0, 0), (0, 128 - d_real)))
    e_pad = idx.shape[0]
    d = table.shape[1]
    nw = 32
    b_per_w = e_pad // nw
    nch = b_per_w // chunk

    @functools.partial(
        pl.kernel,
        mesh=plsc.VectorSubcoreMesh(core_axis_name="c", subcore_axis_name="s"),
        out_type=jax.ShapeDtypeStruct((e_pad, d), jnp.float32),
        scratch_types=[
            pltpu.VMEM((chunk,), jnp.int32),
            pltpu.VMEM((chunk, d), jnp.float32),
            pltpu.SemaphoreType.DMA,
        ],
    )
    def gk(table_hbm, idx_hbm, out_hbm, idx_v, rows_v, sem):
        wid = lax.axis_index("s") * 2 + lax.axis_index("c")
        base = wid * b_per_w

        def body(j, _):
            off = base + j * chunk
            pltpu.sync_copy(idx_hbm.at[pl.ds(off, chunk)], idx_v)
            pltpu.async_copy(table_hbm.at[idx_v], rows_v, sem).wait()
            pltpu.sync_copy(rows_v, out_hbm.at[pl.ds(off, chunk)])
            return 0

        lax.fori_loop(0, nch, body, 0)

    return gk(table, idx)[:, :d_real]


def _xa_kernel(x_ref, wa_ref, a_ref):
    a_ref[...] = _bdot(x_ref[...], wa_ref[...])


def _xa(x, wa, br):
    n, f = x.shape
    h = wa.shape[1]
    return pl.pallas_call(
        _xa_kernel,
        grid=(n // br,),
        in_specs=[pl.BlockSpec((br, f), lambda i: (i, 0)),
                  pl.BlockSpec((f, h), lambda i: (0, 0))],
        out_specs=pl.BlockSpec((br, h), lambda i: (i, 0)),
        out_shape=jax.ShapeDtypeStruct((n, h), jnp.float32),
    )(x, wa)


def _l1_kernel(xi_ref, xj_ref, a_ref, wb_ref, b_ref, pre_ref, s_ref, q_ref,
               *, bn, f, h):
    i = pl.program_id(0)

    @pl.when(i == 0)
    def _():
        s_ref[...] = jnp.zeros_like(s_ref)
        q_ref[...] = jnp.zeros_like(q_ref)

    t = xj_ref[...].reshape(bn, K, f) - xi_ref[...][:, None, :]
    td = _bdot(t.reshape(bn * K, f), wb_ref[...]).reshape(bn, K, h)
    pre = ((a_ref[...][:, None, :] + td) + b_ref[...]).reshape(bn * K, h)
    pre_ref[...] = pre.astype(pre_ref.dtype)
    s_ref[...] += jnp.sum(pre, axis=0, keepdims=True)
    q_ref[...] += jnp.sum(pre * pre, axis=0, keepdims=True)


def _l1(xi, xj, a, wb, b, bn, dt):
    n, h = a.shape
    f = xi.shape[1]
    kern = lambda *r: _l1_kernel(*r, bn=bn, f=f, h=h)
    return pl.pallas_call(
        kern,
        grid=(n // bn,),
        in_specs=[pl.BlockSpec((bn, f), lambda i: (i, 0)),
                  pl.BlockSpec((bn * K, f), lambda i: (i, 0)),
                  pl.BlockSpec((bn, h), lambda i: (i, 0)),
                  pl.BlockSpec((f, h), lambda i: (0, 0)),
                  pl.BlockSpec((1, h), lambda i: (0, 0))],
        out_specs=[pl.BlockSpec((bn * K, h), lambda i: (i, 0)),
                   pl.BlockSpec((1, h), lambda i: (0, 0)),
                   pl.BlockSpec((1, h), lambda i: (0, 0))],
        out_shape=[jax.ShapeDtypeStruct((n * K, h), dt),
                   jax.ShapeDtypeStruct((1, h), jnp.float32),
                   jax.ShapeDtypeStruct((1, h), jnp.float32)],
    )(xi, xj, a, wb, b)


def _l2_kernel(pre_ref, m_ref, den_ref, g_ref, be_ref, w_ref, b_ref,
               o_ref, s_ref, q_ref):
    i = pl.program_id(0)

    @pl.when(i == 0)
    def _():
        s_ref[...] = jnp.zeros_like(s_ref)
        q_ref[...] = jnp.zeros_like(q_ref)

    hact = jnp.maximum(
        (pre_ref[...].astype(jnp.float32) - m_ref[...]) / den_ref[...]
        * g_ref[...] + be_ref[...], 0.0)
    o = _bdot(hact, w_ref[...]) + b_ref[...]
    o_ref[...] = o.astype(o_ref.dtype)
    s_ref[...] += jnp.sum(o, axis=0, keepdims=True)
    q_ref[...] += jnp.sum(o * o, axis=0, keepdims=True)


def _l2(pre, m, den, g, be, w, b, bblk, dt):
    e, h = pre.shape
    ho = w.shape[1]
    small = [pl.BlockSpec((1, h), lambda i: (0, 0))] * 4
    return pl.pallas_call(
        _l2_kernel,
        grid=(e // bblk,),
        in_specs=[pl.BlockSpec((bblk, h), lambda i: (i, 0))] + small
        + [pl.BlockSpec((h, ho), lambda i: (0, 0)),
           pl.BlockSpec((1, ho), lambda i: (0, 0))],
        out_specs=[pl.BlockSpec((bblk, ho), lambda i: (i, 0)),
                   pl.BlockSpec((1, ho), lambda i: (0, 0)),
                   pl.BlockSpec((1, ho), lambda i: (0, 0))],
        out_shape=[jax.ShapeDtypeStruct((e, ho), dt),
                   jax.ShapeDtypeStruct((1, ho), jnp.float32),
                   jax.ShapeDtypeStruct((1, ho), jnp.float32)],
    )(pre, m, den, g, be, w, b)


def _l3_kernel(pre_ref, m_ref, den_ref, g_ref, be_ref, o_ref, sq_ref,
               *, bn, h):
    hact = jnp.maximum(
        (pre_ref[...].astype(jnp.float32) - m_ref[...]) / den_ref[...]
        * g_ref[...] + be_ref[...], 0.0)
    xo = jnp.max(hact.reshape(bn, K, h), axis=1)
    o_ref[...] = xo
    sq_ref[...] = jnp.sum(xo * xo, axis=1, keepdims=True)


def _l3(pre, m, den, g, be, bn):
    e, h = pre.shape
    n = e // K
    small = [pl.BlockSpec((1, h), lambda i: (0, 0))] * 4
    kern = lambda *r: _l3_kernel(*r, bn=bn, h=h)
    return pl.pallas_call(
        kern,
        grid=(n // bn,),
        in_specs=[pl.BlockSpec((bn * K, h), lambda i: (i, 0))] + small,
        out_specs=[pl.BlockSpec((bn, h), lambda i: (i, 0)),
                   pl.BlockSpec((bn, 1), lambda i: (i, 0))],
        out_shape=[jax.ShapeDtypeStruct((n, h), jnp.float32),
                   jax.ShapeDtypeStruct((n, 1), jnp.float32)],
    )(pre, m, den, g, be)


def _stats(s, q, cnt, g, be):
    m = s / cnt
    v = q / cnt - m * m
    den = jnp.sqrt(v + 1e-5)
    return m, den, g.reshape(1, -1), be.reshape(1, -1)


def _edge_conv(x_in, xj, w1, b1, g1, be1, w2, b2, g2, be2, bn,
               dt=jnp.float32):
    n, fin = x_in.shape
    a = _xa(x_in, w1[:fin], 2000)
    pre1, s1, q1 = _l1(x_in, xj, a, w1[fin:], b1.reshape(1, -1), bn, dt)
    pre2, s2, q2 = _l2(pre1, *_stats(s1, q1, n * K, g1, be1),
                       w2, b2.reshape(1, -1), bn * K, dt)
    return _l3(pre2, *_stats(s2, q2, n * K, g2, be2), bn)


def _cls1_kernel(x1_ref, x2_ref, wt_ref, wb_ref, b_ref, o_ref, s_ref, q_ref):
    i = pl.program_id(0)

    @pl.when(i == 0)
    def _():
        s_ref[...] = jnp.zeros_like(s_ref)
        q_ref[...] = jnp.zeros_like(q_ref)

    o = _bdot(x1_ref[...], wt_ref[...]) + _bdot(x2_ref[...], wb_ref[...]) \
        + b_ref[...]
    o_ref[...] = o
    s_ref[...] += jnp.sum(o, axis=0, keepdims=True)
    q_ref[...] += jnp.sum(o * o, axis=0, keepdims=True)


def _cls1(x1, x2, wt, wb, b, br):
    n, h1 = x1.shape
    h2 = x2.shape[1]
    ho = wt.shape[1]
    return pl.pallas_call(
        _cls1_kernel,
        grid=(n // br,),
        in_specs=[pl.BlockSpec((br, h1), lambda i: (i, 0)),
                  pl.BlockSpec((br, h2), lambda i: (i, 0)),
                  pl.BlockSpec((h1, ho), lambda i: (0, 0)),
                  pl.BlockSpec((h2, ho), lambda i: (0, 0)),
                  pl.BlockSpec((1, ho), lambda i: (0, 0))],
        out_specs=[pl.BlockSpec((br, ho), lambda i: (i, 0)),
                   pl.BlockSpec((1, ho), lambda i: (0, 0)),
                   pl.BlockSpec((1, ho), lambda i: (0, 0))],
        out_shape=[jax.ShapeDtypeStruct((n, ho), jnp.float32),
                   jax.ShapeDtypeStruct((1, ho), jnp.float32),
                   jax.ShapeDtypeStruct((1, ho), jnp.float32)],
    )(x1, x2, wt, wb, b)


def _cls2_kernel(pre_ref, m_ref, den_ref, g_ref, be_ref, wT_ref, b_ref, o_ref):
    hact = jnp.maximum(
        (pre_ref[...] - m_ref[...]) / den_ref[...] * g_ref[...] + be_ref[...],
        0.0)
    hb = hact.astype(jnp.bfloat16).astype(jnp.float32)
    wb = wT_ref[...].astype(jnp.bfloat16).astype(jnp.float32)
    o_ref[...] = jnp.sum(hb * wb, axis=1, keepdims=True) + b_ref[...]


def _cls2(pre, m, den, g, be, wT, b, br):
    n, h = pre.shape
    small = [pl.BlockSpec((1, h), lambda i: (0, 0))] * 5
    return pl.pallas_call(
        _cls2_kernel,
        grid=(n // br,),
        in_specs=[pl.BlockSpec((br, h), lambda i: (i, 0))] + small
        + [pl.BlockSpec((1, 1), lambda i: (0, 0))],
        out_specs=pl.BlockSpec((br, 1), lambda i: (i, 0)),
        out_shape=jax.ShapeDtypeStruct((n, 1), jnp.float32),
    )(pre, m, den, g, be, wT, b)


def kernel(x, batch, in_gamma, in_beta, c1_w1, c1_b1, c1_g1, c1_be1,
           c1_w2, c1_b2, c1_g2, c1_be2, c2_w1, c2_b1, c2_g1, c2_be1,
           c2_w2, c2_b2, c2_g2, c2_be2, cl_w1, cl_b1, cl_g, cl_be,
           cl_w2, cl_b2):
    n = x.shape[0]
    bk = 512
    batch = batch.astype(jnp.int32)
    br_p = jnp.pad(batch.reshape(1, n), ((0, 0), (0, NPAD - n)),
                   constant_values=-2)
    bc_p = jnp.pad(batch.reshape(n, 1), ((0, NPAD - n), (0, 0)),
                   constant_values=-1)
    batch_hi = jnp.pad(batch, (0, NPAD - n), constant_values=7)
    lo = jnp.searchsorted(batch, batch_hi[0::bk], side='left').astype(jnp.int32)
    hi = jnp.searchsorted(batch, batch_hi[bk - 1::bk], side='right').astype(jnp.int32)

    padc = lambda a: jnp.pad(a, ((0, 0), (0, NPAD - n)))
    padr = lambda a: jnp.pad(a, ((0, NPAD - n), (0, 0)))

    m0 = jnp.mean(x, axis=0, keepdims=True)
    den0 = jnp.sqrt(jnp.var(x, axis=0, keepdims=True) + 1e-5)
    x0, sq0 = _bn_apply(x, m0, den0, in_gamma.reshape(1, -1),
                        in_beta.reshape(1, -1), 2000)

    idx1 = _knn(padr(x0), padc(x0.T), padc(sq0.reshape(1, n)), padr(sq0),
                br_p, bc_p, lo, hi, bk)
    epad = 204800
    xj1 = _sc_gather(x0, jnp.pad(idx1.reshape(-1), (0, epad - n * K)))[:n * K]
    x1, sq1 = _edge_conv(x0, xj1, c1_w1, c1_b1, c1_g1, c1_be1,
                         c1_w2, c1_b2, c1_g2, c1_be2, 200)

    idx2 = _knn(padr(x1), padc(x1.T), padc(sq1.reshape(1, n)), padr(sq1),
                br_p, bc_p, lo, hi, bk)
    xj2 = _sc_gather(x1, jnp.pad(idx2.reshape(-1), (0, epad - n * K)))[:n * K]

    x2, _ = _edge_conv(x1, xj2, c2_w1, c2_b1, c2_g1, c2_be1,
                       c2_w2, c2_b2, c2_g2, c2_be2, 200, jnp.bfloat16)

    pre, s, q = _cls1(x1, x2, cl_w1[:x1.shape[1]], cl_w1[x1.shape[1]:],
                      cl_b1.reshape(1, -1), 2000)
    return _cls2(pre, *_stats(s, q, n, cl_g, cl_be),
                 cl_w2.reshape(1, -1), cl_b2.reshape(1, 1), 2000)
